# baseline (speedup 1.0000x reference)
import jax
import jax.numpy as jnp
from jax.experimental import pallas as pl

FG_THRESH = 0.9
BG_THRESH = 0.1

_ROWS = 16384
_COLS = 1024
_BLOCK_ROWS = 512


def _body(t_ref, a_ref, o_ref):
    t = t_ref[...]
    a = a_ref[...]
    keep = (t > FG_THRESH) | (t < BG_THRESH)
    o_ref[...] = jnp.where(keep, t, a)


def kernel(trimap, alpha):
    shape = trimap.shape
    t2 = trimap.reshape(_ROWS, _COLS)
    a2 = alpha.reshape(_ROWS, _COLS)
    out = pl.pallas_call(
        _body,
        grid=(_ROWS // _BLOCK_ROWS,),
        in_specs=[
            pl.BlockSpec((_BLOCK_ROWS, _COLS), lambda i: (i, 0)),
            pl.BlockSpec((_BLOCK_ROWS, _COLS), lambda i: (i, 0)),
        ],
        out_specs=pl.BlockSpec((_BLOCK_ROWS, _COLS), lambda i: (i, 0)),
        out_shape=jax.ShapeDtypeStruct((_ROWS, _COLS), jnp.float32),
    )(t2, a2)
    return out.reshape(shape)


# speedup vs baseline: 4.4261x; 4.4261x over previous
import jax
import jax.numpy as jnp
from jax.experimental import pallas as pl

FG_THRESH = 0.9
BG_THRESH = 0.1

_ROWS = 32768
_COLS = 512
_BLOCK_ROWS = 2048


def _body(t_ref, a_ref, o_ref):
    t = t_ref[...]
    a = a_ref[...]
    keep = (t > FG_THRESH) | (t < BG_THRESH)
    o_ref[...] = jnp.where(keep, t, a)


def kernel(trimap, alpha):
    shape = trimap.shape
    t2 = trimap.reshape(_ROWS, _COLS)
    a2 = alpha.reshape(_ROWS, _COLS)
    out = pl.pallas_call(
        _body,
        grid=(_ROWS // _BLOCK_ROWS,),
        in_specs=[
            pl.BlockSpec((_BLOCK_ROWS, _COLS), lambda i: (i, 0)),
            pl.BlockSpec((_BLOCK_ROWS, _COLS), lambda i: (i, 0)),
        ],
        out_specs=pl.BlockSpec((_BLOCK_ROWS, _COLS), lambda i: (i, 0)),
        out_shape=jax.ShapeDtypeStruct((_ROWS, _COLS), jnp.float32),
    )(t2, a2)
    return out.reshape(shape)
